# Initial kernel scaffold; baseline (speedup 1.0000x reference)
import jax
import jax.numpy as jnp
from jax.experimental import pallas as pl


def kernel(x_nchw, base_0_w_mat, base_0_bias, base_1_w_mat, base_1_bias, extras_0_w_mat, extras_0_bias, extras_1_w_mat, extras_1_bias, extras_2_w_mat, extras_2_bias, extras_3_w_mat, extras_3_bias, extras_4_w_mat, extras_4_bias, extras_5_w_mat, extras_5_bias, extras_6_w_mat, extras_6_bias, extras_7_w_mat, extras_7_bias, heads_0_w_mat, heads_0_bias, heads_1_w_mat, heads_1_bias, heads_2_w_mat, heads_2_bias, heads_3_w_mat, heads_3_bias, heads_4_w_mat, heads_4_bias, heads_5_w_mat, heads_5_bias):
    raise NotImplementedError("write your pallas kernel here")


# speedup vs baseline: 4.1846x; 4.1846x over previous
import functools

import jax
import jax.numpy as jnp
from jax.experimental import pallas as pl
from jax.experimental.pallas import tpu as pltpu

_VMEM_LIMIT = 48 * 1024 * 1024


def _ru(x, m):
    return ((x + m - 1) // m) * m


def _taps_kernel(x_ref, w_ref, b_ref, o_ref, *, offsets, cin, m_out, relu):
    acc = None
    for t, off in enumerate(offsets):
        part = jnp.dot(x_ref[pl.ds(off, m_out), :],
                       w_ref[pl.ds(t * cin, cin), :],
                       preferred_element_type=jnp.float32)
        acc = part if acc is None else acc + part
    y = acc + b_ref[...]
    if relu:
        y = jnp.maximum(y, 0.0)
    o_ref[...] = y.astype(o_ref.dtype)


def _mm_kernel(x_ref, w_ref, b_ref, o_ref, *, relu):
    y = jnp.dot(x_ref[...], w_ref[...],
                preferred_element_type=jnp.float32) + b_ref[...]
    if relu:
        y = jnp.maximum(y, 0.0)
    o_ref[...] = y.astype(o_ref.dtype)


def _fused_conv(xp, w_mat, bias, offsets, m_out, relu, out_dtype):
    B, R, C = xp.shape
    Np = w_mat.shape[1]
    TN = Np if Np <= 512 else 512
    out = pl.pallas_call(
        functools.partial(_taps_kernel, offsets=offsets, cin=C,
                          m_out=m_out, relu=relu),
        out_shape=jax.ShapeDtypeStruct((B, m_out, Np), out_dtype),
        grid=(Np // TN, B),
        in_specs=[
            pl.BlockSpec((None, R, C), lambda j, b: (b, 0, 0)),
            pl.BlockSpec((len(offsets) * C, TN), lambda j, b: (0, j)),
            pl.BlockSpec((1, TN), lambda j, b: (0, j)),
        ],
        out_specs=pl.BlockSpec((None, m_out, TN), lambda j, b: (b, 0, j)),
        compiler_params=pltpu.CompilerParams(
            dimension_semantics=("parallel", "parallel"),
            vmem_limit_bytes=_VMEM_LIMIT),
    )(xp, w_mat, bias)
    return out


def _matmul_bias(x, w_mat, bias, relu, out_dtype):
    M, K = x.shape
    Np = w_mat.shape[1]
    if M >= 4096:
        TM = 512
    elif M >= 128:
        TM = _ru((M + 1) // 2, 16)
    else:
        TM = _ru(M, 16)
    Mp = _ru(M, TM)
    if Mp != M:
        x = jnp.pad(x, ((0, Mp - M), (0, 0)))
    out = pl.pallas_call(
        functools.partial(_mm_kernel, relu=relu),
        out_shape=jax.ShapeDtypeStruct((Mp, Np), out_dtype),
        grid=(Mp // TM,),
        in_specs=[
            pl.BlockSpec((TM, K), lambda i: (i, 0)),
            pl.BlockSpec((K, Np), lambda i: (0, 0)),
            pl.BlockSpec((1, Np), lambda i: (0, 0)),
        ],
        out_specs=pl.BlockSpec((TM, Np), lambda i: (i, 0)),
        compiler_params=pltpu.CompilerParams(
            dimension_semantics=("parallel",),
            vmem_limit_bytes=_VMEM_LIMIT),
    )(x, w_mat, bias)
    if Mp != M:
        out = out[:M]
    return out


def _plane_s1(x, pad):
    B, H, W, C = x.shape
    OH, OW = H + 2 * pad - 2, W + 2 * pad - 2
    Wp = _ru(W + 2 * pad, 8)
    rows = (H + 2 * pad) * Wp
    R = _ru(rows + 8, 16)
    xp = jnp.pad(x, ((0, 0), (pad, pad), (pad, Wp - W - pad), (0, 0)))
    xp = xp.reshape(B, rows, C)
    xp = jnp.pad(xp, ((0, 0), (0, R - rows), (0, 0)))
    offsets = tuple(di * Wp + dj for di in range(3) for dj in range(3))
    return xp, offsets, OH * Wp, OH, OW, Wp


def _parity_s2(x):
    B, H, W, C = x.shape
    OH, OW = (H - 1) // 2 + 1, (W - 1) // 2 + 1
    He, We = _ru(H + 2, 2), _ru(W + 2, 2)
    PH, PW = He // 2, We // 2
    PWp = _ru(PW, 8)
    xp = jnp.pad(x, ((0, 0), (1, He - H - 1), (1, We - W - 1), (0, 0)))
    xp = xp.reshape(B, PH, 2, PW, 2, C).transpose(0, 2, 4, 1, 3, 5)
    xp = jnp.pad(xp, ((0, 0), (0, 0), (0, 0), (0, 0), (0, PWp - PW), (0, 0)))
    S = PH * PWp
    xp = xp.reshape(B, 4 * S, C)
    R = _ru(4 * S + 8, 16)
    xp = jnp.pad(xp, ((0, 0), (0, R - 4 * S), (0, 0)))
    offsets = tuple(((di % 2) * 2 + (dj % 2)) * S + (di // 2) * PWp + (dj // 2)
                    for di in range(3) for dj in range(3))
    return xp, offsets, OH * PWp, OH, OW, PWp


def _im2col3(x, stride, pad):
    B, H, W, C = x.shape
    if pad:
        x = jnp.pad(x, ((0, 0), (pad, pad), (pad, pad), (0, 0)))
    OH = (H + 2 * pad - 3) // stride + 1
    OW = (W + 2 * pad - 3) // stride + 1
    cols = [x[:, i:i + stride * OH:stride, j:j + stride * OW:stride, :]
            for i in range(3) for j in range(3)]
    return jnp.concatenate(cols, -1).reshape(B * OH * OW, 9 * C), OH, OW


def _conv_fused_s1(x, w_mat, bias, relu, out_dtype, n):
    xp, offs, m_out, OH, OW, Wp = _plane_s1(x, pad=1)
    y = _fused_conv(xp, w_mat, bias, offs, m_out, relu, out_dtype)
    B = x.shape[0]
    return y.reshape(B, OH, Wp, -1)[:, :, :OW, :n]


def _conv_fused_s2(x, w_mat, bias, relu, out_dtype, n):
    xp, offs, m_out, OH, OW, PWp = _parity_s2(x)
    y = _fused_conv(xp, w_mat, bias, offs, m_out, relu, out_dtype)
    B = x.shape[0]
    return y.reshape(B, OH, PWp, -1)[:, :, :OW, :n]


def _conv_mm(x, w_mat, bias, relu, out_dtype, n, stride=1, pad=0, k=3):
    B = x.shape[0]
    if k == 1:
        patches = x.reshape(-1, x.shape[-1])
        OH, OW = x.shape[1], x.shape[2]
    else:
        patches, OH, OW = _im2col3(x, stride, pad)
    y = _matmul_bias(patches, w_mat, bias, relu, out_dtype)
    return y.reshape(B, OH, OW, -1)[..., :n]


def kernel(x_nchw, base_0_w_mat, base_0_bias, base_1_w_mat, base_1_bias,
           extras_0_w_mat, extras_0_bias, extras_1_w_mat, extras_1_bias,
           extras_2_w_mat, extras_2_bias, extras_3_w_mat, extras_3_bias,
           extras_4_w_mat, extras_4_bias, extras_5_w_mat, extras_5_bias,
           extras_6_w_mat, extras_6_bias, extras_7_w_mat, extras_7_bias,
           heads_0_w_mat, heads_0_bias, heads_1_w_mat, heads_1_bias,
           heads_2_w_mat, heads_2_bias, heads_3_w_mat, heads_3_bias,
           heads_4_w_mat, heads_4_bias, heads_5_w_mat, heads_5_bias):
    B = x_nchw.shape[0]
    x = jnp.transpose(x_nchw, (0, 2, 3, 1)).astype(jnp.bfloat16)

    H, W = x.shape[1], x.shape[2]
    xpad = jnp.pad(x, ((0, 0), (1, 1), (1, 1), (0, 0)))
    patches = jnp.concatenate(
        [xpad[:, i:i + H, j:j + W, :] for i in range(3) for j in range(3)], -1)
    y0 = jnp.dot(patches.reshape(B * H * W, 27), base_0_w_mat,
                 preferred_element_type=jnp.float32) + base_0_bias
    fm0 = jnp.maximum(y0, 0.0).astype(jnp.bfloat16).reshape(B, H, W, 512)

    fm1 = _conv_fused_s2(fm0, base_1_w_mat, base_1_bias, True, jnp.bfloat16, 1024)

    e0 = _conv_mm(fm1, extras_0_w_mat, extras_0_bias, True, jnp.bfloat16, 256, k=1)
    s2 = _conv_fused_s2(e0, extras_1_w_mat, extras_1_bias, True, jnp.bfloat16, 512)
    e2 = _conv_mm(s2, extras_2_w_mat, extras_2_bias, True, jnp.bfloat16, 128, k=1)
    s3 = _conv_mm(e2, extras_3_w_mat, extras_3_bias, True, jnp.bfloat16, 256,
                  stride=2, pad=1)
    e4 = _conv_mm(s3, extras_4_w_mat, extras_4_bias, True, jnp.bfloat16, 128, k=1)
    s4 = _conv_mm(e4, extras_5_w_mat, extras_5_bias, True, jnp.bfloat16, 256,
                  stride=1, pad=0)
    e6 = _conv_mm(s4, extras_6_w_mat, extras_6_bias, True, jnp.bfloat16, 128, k=1)
    s5 = _conv_mm(e6, extras_7_w_mat, extras_7_bias, True, jnp.bfloat16, 256,
                  stride=1, pad=0)

    h0 = _conv_fused_s1(fm0, heads_0_w_mat, heads_0_bias, False, jnp.float32, 48)
    h1 = _conv_fused_s1(fm1, heads_1_w_mat, heads_1_bias, False, jnp.float32, 48)
    h2 = _conv_fused_s1(s2, heads_2_w_mat, heads_2_bias, False, jnp.float32, 48)
    h3 = _conv_mm(s3, heads_3_w_mat, heads_3_bias, False, jnp.float32, 48,
                  stride=1, pad=1)
    h4 = _conv_mm(s4, heads_4_w_mat, heads_4_bias, False, jnp.float32, 32,
                  stride=1, pad=1)
    h5 = _matmul_bias(s5.reshape(B, 256), heads_5_w_mat[4 * 256:5 * 256, :],
                      heads_5_bias, False, jnp.float32)
    h5 = h5.reshape(B, 1, 1, -1)[..., :32]

    nls = (24, 24, 24, 24, 16, 16)
    locs, confs = [], []
    for y, nl in zip((h0, h1, h2, h3, h4, h5), nls):
        locs.append(y[..., :nl].reshape(B, -1))
        confs.append(y[..., nl:2 * nl].reshape(B, -1))
    loc = jnp.concatenate(locs, axis=1).reshape(B, -1, 4)
    conf = jnp.concatenate(confs, axis=1).reshape(B, -1, 4)
    return loc, conf


# speedup vs baseline: 5.1392x; 1.2281x over previous
import functools

import jax
import jax.numpy as jnp
from jax.experimental import pallas as pl
from jax.experimental.pallas import tpu as pltpu

_VMEM_LIMIT = 48 * 1024 * 1024


def _ru(x, m):
    return ((x + m - 1) // m) * m


def _taps_kernel(x_ref, w_ref, b_ref, o_ref, *, offsets, cin, m_out, relu,
                 pad_out=None):
    acc = None
    for t, off in enumerate(offsets):
        part = jnp.dot(x_ref[pl.ds(off, m_out), :],
                       w_ref[pl.ds(t * cin, cin), :],
                       preferred_element_type=jnp.float32)
        acc = part if acc is None else acc + part
    y = acc + b_ref[...]
    if relu:
        y = jnp.maximum(y, 0.0)
    if pad_out is None:
        o_ref[...] = y.astype(o_ref.dtype)
    else:
        shift, wp, wlim = pad_out
        i = jax.lax.broadcasted_iota(jnp.int32, (m_out, 1), 0)
        ow = i - (i // wp) * wp
        y = jnp.where(ow < wlim, y, 0.0)
        o_ref[...] = jnp.zeros(o_ref.shape, o_ref.dtype)
        o_ref[pl.ds(shift, m_out), :] = y.astype(o_ref.dtype)


def _mm_kernel(x_ref, w_ref, b_ref, o_ref, *, relu):
    y = jnp.dot(x_ref[...], w_ref[...],
                preferred_element_type=jnp.float32) + b_ref[...]
    if relu:
        y = jnp.maximum(y, 0.0)
    o_ref[...] = y.astype(o_ref.dtype)


def _fused_conv(xp, w_mat, bias, offsets, m_out, relu, out_dtype,
                pad_out=None):
    B, R, C = xp.shape
    Np = w_mat.shape[1]
    TN = Np if Np <= 1024 else 512
    rows_out = m_out if pad_out is None else pad_out[0]
    po = None if pad_out is None else pad_out[1:]
    out = pl.pallas_call(
        functools.partial(_taps_kernel, offsets=offsets, cin=C,
                          m_out=m_out, relu=relu, pad_out=po),
        out_shape=jax.ShapeDtypeStruct((B, rows_out, Np), out_dtype),
        grid=(Np // TN, B),
        in_specs=[
            pl.BlockSpec((None, R, C), lambda j, b: (b, 0, 0)),
            pl.BlockSpec((len(offsets) * C, TN), lambda j, b: (0, j)),
            pl.BlockSpec((1, TN), lambda j, b: (0, j)),
        ],
        out_specs=pl.BlockSpec((None, rows_out, TN), lambda j, b: (b, 0, j)),
        compiler_params=pltpu.CompilerParams(
            dimension_semantics=("parallel", "parallel"),
            vmem_limit_bytes=_VMEM_LIMIT),
    )(xp, w_mat, bias)
    return out


def _matmul_bias(x, w_mat, bias, relu, out_dtype):
    M, K = x.shape
    Np = w_mat.shape[1]
    if M >= 4096:
        TM = 512
        for c in (512, 448, 384, 320, 256):
            if M % c == 0:
                TM = c
                break
    elif M >= 128:
        TM = _ru((M + 1) // 2, 16)
    else:
        TM = _ru(M, 16)
    Mp = _ru(M, TM)
    if Mp != M:
        x = jnp.pad(x, ((0, Mp - M), (0, 0)))
    out = pl.pallas_call(
        functools.partial(_mm_kernel, relu=relu),
        out_shape=jax.ShapeDtypeStruct((Mp, Np), out_dtype),
        grid=(Mp // TM,),
        in_specs=[
            pl.BlockSpec((TM, K), lambda i: (i, 0)),
            pl.BlockSpec((K, Np), lambda i: (0, 0)),
            pl.BlockSpec((1, Np), lambda i: (0, 0)),
        ],
        out_specs=pl.BlockSpec((TM, Np), lambda i: (i, 0)),
        compiler_params=pltpu.CompilerParams(
            dimension_semantics=("parallel",),
            vmem_limit_bytes=_VMEM_LIMIT),
    )(x, w_mat, bias)
    if Mp != M:
        out = out[:M]
    return out


def _s1_offsets(Wp):
    return tuple(di * Wp + dj for di in range(3) for dj in range(3))


def _parity_planes(xpad):
    B, He, We, C = xpad.shape
    PH, PW = He // 2, We // 2
    PWp = _ru(PW, 8)
    xp = xpad.reshape(B, PH, 2, PW, 2, C).transpose(0, 2, 4, 1, 3, 5)
    xp = jnp.pad(xp, ((0, 0), (0, 0), (0, 0), (0, 0), (0, PWp - PW), (0, 0)))
    S = PH * PWp
    xp = xp.reshape(B, 4 * S, C)
    offsets = tuple(((di % 2) * 2 + (dj % 2)) * S + (di // 2) * PWp + (dj // 2)
                    for di in range(3) for dj in range(3))
    return xp, offsets, PWp


def _im2col3(x, stride, pad):
    B, H, W, C = x.shape
    if pad:
        x = jnp.pad(x, ((0, 0), (pad, pad), (pad, pad), (0, 0)))
    OH = (H + 2 * pad - 3) // stride + 1
    OW = (W + 2 * pad - 3) // stride + 1
    cols = [x[:, i:i + stride * OH:stride, j:j + stride * OW:stride, :]
            for i in range(3) for j in range(3)]
    return jnp.concatenate(cols, -1).reshape(B * OH * OW, 9 * C), OH, OW


def _conv_mm(x, w_mat, bias, relu, out_dtype, n, stride=1, pad=0, k=3):
    B = x.shape[0]
    if k == 1:
        patches = x.reshape(-1, x.shape[-1])
        OH, OW = x.shape[1], x.shape[2]
    else:
        patches, OH, OW = _im2col3(x, stride, pad)
    y = _matmul_bias(patches, w_mat, bias, relu, out_dtype)
    return y.reshape(B, OH, OW, -1)[..., :n]


def _head_unpack(y, OH, OW, Wp, nl):
    B, m, _ = y.shape
    y = y[..., :2 * nl]
    y = jnp.pad(y, ((0, 0), (0, OH * Wp - m), (0, 0)))
    y = y.reshape(B, OH, Wp, 2 * nl)[:, :, :OW, :]
    return y[..., :nl].reshape(B, -1), y[..., nl:].reshape(B, -1)


def kernel(x_nchw, base_0_w_mat, base_0_bias, base_1_w_mat, base_1_bias,
           extras_0_w_mat, extras_0_bias, extras_1_w_mat, extras_1_bias,
           extras_2_w_mat, extras_2_bias, extras_3_w_mat, extras_3_bias,
           extras_4_w_mat, extras_4_bias, extras_5_w_mat, extras_5_bias,
           extras_6_w_mat, extras_6_bias, extras_7_w_mat, extras_7_bias,
           heads_0_w_mat, heads_0_bias, heads_1_w_mat, heads_1_bias,
           heads_2_w_mat, heads_2_bias, heads_3_w_mat, heads_3_bias,
           heads_4_w_mat, heads_4_bias, heads_5_w_mat, heads_5_bias):
    B = x_nchw.shape[0]
    x = jnp.transpose(x_nchw, (0, 2, 3, 1)).astype(jnp.bfloat16)

    H, W = x.shape[1], x.shape[2]
    Hp, Wp0 = H + 2, W + 2
    xpad = jnp.pad(x, ((0, 0), (2, 2), (2, 2), (0, 0)))
    patches = jnp.concatenate(
        [xpad[:, i:i + Hp, j:j + Wp0, :] for i in range(3) for j in range(3)],
        -1)
    y0 = jnp.dot(patches.reshape(B * Hp * Wp0, 27), base_0_w_mat,
                 preferred_element_type=jnp.float32) + base_0_bias
    row = jnp.arange(Hp, dtype=jnp.int32)
    edge = ((row > 0) & (row < Hp - 1)).astype(jnp.float32)
    mask = (edge[:, None] * edge[None, :]).reshape(1, Hp, Wp0, 1)
    fm0p = (jnp.maximum(y0, 0.0).reshape(B, Hp, Wp0, 512) *
            mask).astype(jnp.bfloat16)

    xp1, offs1, PWp1 = _parity_planes(fm0p)
    m1 = (19 - 1) * 24 + 19
    fm1p = _fused_conv(xp1, base_1_w_mat, base_1_bias, offs1, m1, True,
                       jnp.bfloat16, pad_out=(21 * 24, 24 + 1, 24, 19))

    e0 = _matmul_bias(fm1p.reshape(B * 504, 1024), extras_0_w_mat,
                      extras_0_bias, True, jnp.bfloat16)
    e0 = e0.reshape(B, 21, 24, 256)[:, 1:20, 1:20, :]
    e0s = jnp.pad(e0, ((0, 0), (1, 2), (1, 2), (0, 0)))

    xp2, offs2, PWp2 = _parity_planes(e0s)
    m2 = (10 - 1) * 16 + 10
    s2p = _fused_conv(xp2, extras_1_w_mat, extras_1_bias, offs2, m2, True,
                      jnp.bfloat16, pad_out=(12 * 16, 16 + 1, 16, 10))

    e2 = _matmul_bias(s2p.reshape(B * 192, 512), extras_2_w_mat,
                      extras_2_bias, True, jnp.bfloat16)
    e2s = e2.reshape(B, 12, 16, 128)[:, 1:11, 1:11, :]

    s3 = _conv_mm(e2s, extras_3_w_mat, extras_3_bias, True, jnp.bfloat16, 256,
                  stride=2, pad=1)
    e4 = _conv_mm(s3, extras_4_w_mat, extras_4_bias, True, jnp.bfloat16, 128,
                  k=1)
    s4 = _conv_mm(e4, extras_5_w_mat, extras_5_bias, True, jnp.bfloat16, 256,
                  stride=1, pad=0)
    e6 = _conv_mm(s4, extras_6_w_mat, extras_6_bias, True, jnp.bfloat16, 128,
                  k=1)
    s5 = _conv_mm(e6, extras_7_w_mat, extras_7_bias, True, jnp.bfloat16, 256,
                  stride=1, pad=0)

    h0 = _fused_conv(fm0p.reshape(B, Hp * Wp0, 512), heads_0_w_mat,
                     heads_0_bias, _s1_offsets(40), (38 - 1) * 40 + 38,
                     False, jnp.float32)
    h1 = _fused_conv(fm1p, heads_1_w_mat, heads_1_bias,
                     _s1_offsets(24), m1, False, jnp.float32)
    h2 = _fused_conv(s2p, heads_2_w_mat, heads_2_bias,
                     _s1_offsets(16), m2, False, jnp.float32)
    h3 = _conv_mm(s3, heads_3_w_mat, heads_3_bias, False, jnp.float32, 48,
                  stride=1, pad=1)
    h4 = _conv_mm(s4, heads_4_w_mat, heads_4_bias, False, jnp.float32, 32,
                  stride=1, pad=1)
    h5 = _matmul_bias(s5.reshape(B, 256), heads_5_w_mat[4 * 256:5 * 256, :],
                      heads_5_bias, False, jnp.float32)

    locs, confs = [], []
    for args in ((h0, 38, 38, 40, 24), (h1, 19, 19, 24, 24),
                 (h2, 10, 10, 16, 24)):
        l, c = _head_unpack(*args)
        locs.append(l)
        confs.append(c)
    for y, nl in ((h3, 24), (h4, 16)):
        locs.append(y[..., :nl].reshape(B, -1))
        confs.append(y[..., nl:2 * nl].reshape(B, -1))
    locs.append(h5[:, :16].reshape(B, -1))
    confs.append(h5[:, 16:32].reshape(B, -1))
    loc = jnp.concatenate(locs, axis=1).reshape(B, -1, 4)
    conf = jnp.concatenate(confs, axis=1).reshape(B, -1, 4)
    return loc, conf


# speedup vs baseline: 5.2125x; 1.0143x over previous
import functools

import jax
import jax.numpy as jnp
from jax.experimental import pallas as pl
from jax.experimental.pallas import tpu as pltpu

_VMEM_LIMIT = 48 * 1024 * 1024


def _ru(x, m):
    return ((x + m - 1) // m) * m


def _taps_kernel(x_ref, w_ref, b_ref, o_ref, *, offsets, cin, m_out, relu,
                 pad_out=None):
    acc = None
    for t, off in enumerate(offsets):
        part = jnp.dot(x_ref[pl.ds(off, m_out), :],
                       w_ref[pl.ds(t * cin, cin), :],
                       preferred_element_type=jnp.float32)
        acc = part if acc is None else acc + part
    y = acc + b_ref[...]
    if relu:
        y = jnp.maximum(y, 0.0)
    if pad_out is None:
        o_ref[...] = y.astype(o_ref.dtype)
    else:
        shift, wp, wlim = pad_out
        i = jax.lax.broadcasted_iota(jnp.int32, (m_out, 1), 0)
        ow = i - (i // wp) * wp
        y = jnp.where(ow < wlim, y, 0.0)
        o_ref[...] = jnp.zeros(o_ref.shape, o_ref.dtype)
        o_ref[pl.ds(shift, m_out), :] = y.astype(o_ref.dtype)


def _packed_taps_kernel(x_ref, w_ref, b_ref, o_ref, *, offsets, m_out):
    mread = offsets[-1] + m_out
    p = jnp.dot(x_ref[pl.ds(0, mread), :], w_ref[...],
                preferred_element_type=jnp.float32)
    acc = None
    for t, off in enumerate(offsets):
        sl = p[off:off + m_out, t * 128:(t + 1) * 128]
        acc = sl if acc is None else acc + sl
    o_ref[...] = (acc + b_ref[...]).astype(o_ref.dtype)


def _mm_kernel(x_ref, w_ref, b_ref, o_ref, *, relu):
    y = jnp.dot(x_ref[...], w_ref[...],
                preferred_element_type=jnp.float32) + b_ref[...]
    if relu:
        y = jnp.maximum(y, 0.0)
    o_ref[...] = y.astype(o_ref.dtype)


def _fused_conv(xp, w_mat, bias, offsets, m_out, relu, out_dtype,
                pad_out=None):
    B, R, C = xp.shape
    Np = w_mat.shape[1]
    TN = Np if Np <= 1024 else 512
    rows_out = m_out if pad_out is None else pad_out[0]
    po = None if pad_out is None else pad_out[1:]
    out = pl.pallas_call(
        functools.partial(_taps_kernel, offsets=offsets, cin=C,
                          m_out=m_out, relu=relu, pad_out=po),
        out_shape=jax.ShapeDtypeStruct((B, rows_out, Np), out_dtype),
        grid=(Np // TN, B),
        in_specs=[
            pl.BlockSpec((None, R, C), lambda j, b: (b, 0, 0)),
            pl.BlockSpec((len(offsets) * C, TN), lambda j, b: (0, j)),
            pl.BlockSpec((1, TN), lambda j, b: (0, j)),
        ],
        out_specs=pl.BlockSpec((None, rows_out, TN), lambda j, b: (b, 0, j)),
        compiler_params=pltpu.CompilerParams(
            dimension_semantics=("parallel", "parallel"),
            vmem_limit_bytes=_VMEM_LIMIT),
    )(xp, w_mat, bias)
    return out


def _packed_head(xp, w_mat, bias, Wp, m_out):
    B, R, C = xp.shape
    w_pack = w_mat.reshape(9, C, 128).transpose(1, 0, 2).reshape(C, 9 * 128)
    offsets = _s1_offsets(Wp)
    out = pl.pallas_call(
        functools.partial(_packed_taps_kernel, offsets=offsets, m_out=m_out),
        out_shape=jax.ShapeDtypeStruct((B, m_out, 128), jnp.float32),
        grid=(B,),
        in_specs=[
            pl.BlockSpec((None, R, C), lambda b: (b, 0, 0)),
            pl.BlockSpec((C, 9 * 128), lambda b: (0, 0)),
            pl.BlockSpec((1, 128), lambda b: (0, 0)),
        ],
        out_specs=pl.BlockSpec((None, m_out, 128), lambda b: (b, 0, 0)),
        compiler_params=pltpu.CompilerParams(
            dimension_semantics=("parallel",),
            vmem_limit_bytes=_VMEM_LIMIT),
    )(xp, w_pack, bias)
    return out


def _matmul_bias(x, w_mat, bias, relu, out_dtype):
    M, K = x.shape
    Np = w_mat.shape[1]
    if M >= 4096:
        TM = 512
        for c in (512, 448, 384, 320, 256):
            if M % c == 0:
                TM = c
                break
    elif M >= 128:
        TM = _ru((M + 1) // 2, 16)
    else:
        TM = _ru(M, 16)
    Mp = _ru(M, TM)
    if Mp != M:
        x = jnp.pad(x, ((0, Mp - M), (0, 0)))
    out = pl.pallas_call(
        functools.partial(_mm_kernel, relu=relu),
        out_shape=jax.ShapeDtypeStruct((Mp, Np), out_dtype),
        grid=(Mp // TM,),
        in_specs=[
            pl.BlockSpec((TM, K), lambda i: (i, 0)),
            pl.BlockSpec((K, Np), lambda i: (0, 0)),
            pl.BlockSpec((1, Np), lambda i: (0, 0)),
        ],
        out_specs=pl.BlockSpec((TM, Np), lambda i: (i, 0)),
        compiler_params=pltpu.CompilerParams(
            dimension_semantics=("parallel",),
            vmem_limit_bytes=_VMEM_LIMIT),
    )(x, w_mat, bias)
    if Mp != M:
        out = out[:M]
    return out


def _s1_offsets(Wp):
    return tuple(di * Wp + dj for di in range(3) for dj in range(3))


def _parity_planes(xpad):
    B, He, We, C = xpad.shape
    PH, PW = He // 2, We // 2
    PWp = _ru(PW, 8)
    xp = xpad.reshape(B, PH, 2, PW, 2, C).transpose(0, 2, 4, 1, 3, 5)
    xp = jnp.pad(xp, ((0, 0), (0, 0), (0, 0), (0, 0), (0, PWp - PW), (0, 0)))
    S = PH * PWp
    xp = xp.reshape(B, 4 * S, C)
    offsets = tuple(((di % 2) * 2 + (dj % 2)) * S + (di // 2) * PWp + (dj // 2)
                    for di in range(3) for dj in range(3))
    return xp, offsets, PWp


def _im2col3(x, stride, pad):
    B, H, W, C = x.shape
    if pad:
        x = jnp.pad(x, ((0, 0), (pad, pad), (pad, pad), (0, 0)))
    OH = (H + 2 * pad - 3) // stride + 1
    OW = (W + 2 * pad - 3) // stride + 1
    cols = [x[:, i:i + stride * OH:stride, j:j + stride * OW:stride, :]
            for i in range(3) for j in range(3)]
    return jnp.concatenate(cols, -1).reshape(B * OH * OW, 9 * C), OH, OW


def _conv_mm(x, w_mat, bias, relu, out_dtype, n, stride=1, pad=0, k=3):
    B = x.shape[0]
    if k == 1:
        patches = x.reshape(-1, x.shape[-1])
        OH, OW = x.shape[1], x.shape[2]
    else:
        patches, OH, OW = _im2col3(x, stride, pad)
    y = _matmul_bias(patches, w_mat, bias, relu, out_dtype)
    return y.reshape(B, OH, OW, -1)[..., :n]


def _head_unpack(y, OH, OW, Wp, nl):
    B, m, _ = y.shape
    y = y[..., :2 * nl]
    y = jnp.pad(y, ((0, 0), (0, OH * Wp - m), (0, 0)))
    y = y.reshape(B, OH, Wp, 2 * nl)[:, :, :OW, :]
    return y[..., :nl].reshape(B, -1), y[..., nl:].reshape(B, -1)


def kernel(x_nchw, base_0_w_mat, base_0_bias, base_1_w_mat, base_1_bias,
           extras_0_w_mat, extras_0_bias, extras_1_w_mat, extras_1_bias,
           extras_2_w_mat, extras_2_bias, extras_3_w_mat, extras_3_bias,
           extras_4_w_mat, extras_4_bias, extras_5_w_mat, extras_5_bias,
           extras_6_w_mat, extras_6_bias, extras_7_w_mat, extras_7_bias,
           heads_0_w_mat, heads_0_bias, heads_1_w_mat, heads_1_bias,
           heads_2_w_mat, heads_2_bias, heads_3_w_mat, heads_3_bias,
           heads_4_w_mat, heads_4_bias, heads_5_w_mat, heads_5_bias):
    B = x_nchw.shape[0]
    x = jnp.transpose(x_nchw, (0, 2, 3, 1)).astype(jnp.bfloat16)

    H, W = x.shape[1], x.shape[2]
    Hp, Wp0 = H + 2, W + 2
    xpad = jnp.pad(x, ((0, 0), (2, 2), (2, 2), (0, 0)))
    patches = jnp.concatenate(
        [xpad[:, i:i + Hp, j:j + Wp0, :] for i in range(3) for j in range(3)],
        -1)
    y0 = jnp.dot(patches.reshape(B * Hp * Wp0, 27), base_0_w_mat,
                 preferred_element_type=jnp.float32) + base_0_bias
    row = jnp.arange(Hp, dtype=jnp.int32)
    edge = ((row > 0) & (row < Hp - 1)).astype(jnp.float32)
    mask = (edge[:, None] * edge[None, :]).reshape(1, Hp, Wp0, 1)
    fm0p = (jnp.maximum(y0, 0.0).reshape(B, Hp, Wp0, 512) *
            mask).astype(jnp.bfloat16)

    xp1, offs1, PWp1 = _parity_planes(fm0p)
    m1 = (19 - 1) * 24 + 19
    fm1p = _fused_conv(xp1, base_1_w_mat, base_1_bias, offs1, m1, True,
                       jnp.bfloat16, pad_out=(21 * 24, 24 + 1, 24, 19))

    e0 = _matmul_bias(fm1p.reshape(B * 504, 1024), extras_0_w_mat,
                      extras_0_bias, True, jnp.bfloat16)
    e0 = e0.reshape(B, 21, 24, 256)[:, 1:20, 1:20, :]
    e0s = jnp.pad(e0, ((0, 0), (1, 2), (1, 2), (0, 0)))

    xp2, offs2, PWp2 = _parity_planes(e0s)
    m2 = (10 - 1) * 16 + 10
    s2p = _fused_conv(xp2, extras_1_w_mat, extras_1_bias, offs2, m2, True,
                      jnp.bfloat16, pad_out=(12 * 16, 16 + 1, 16, 10))

    e2 = _matmul_bias(s2p.reshape(B * 192, 512), extras_2_w_mat,
                      extras_2_bias, True, jnp.bfloat16)
    e2s = e2.reshape(B, 12, 16, 128)[:, 1:11, 1:11, :]

    s3 = _conv_mm(e2s, extras_3_w_mat, extras_3_bias, True, jnp.bfloat16, 256,
                  stride=2, pad=1)
    e4 = _conv_mm(s3, extras_4_w_mat, extras_4_bias, True, jnp.bfloat16, 128,
                  k=1)
    s4 = _conv_mm(e4, extras_5_w_mat, extras_5_bias, True, jnp.bfloat16, 256,
                  stride=1, pad=0)
    e6 = _conv_mm(s4, extras_6_w_mat, extras_6_bias, True, jnp.bfloat16, 128,
                  k=1)
    s5 = _conv_mm(e6, extras_7_w_mat, extras_7_bias, True, jnp.bfloat16, 256,
                  stride=1, pad=0)

    h0 = _packed_head(fm0p.reshape(B, Hp * Wp0, 512), heads_0_w_mat,
                      heads_0_bias, 40, (38 - 1) * 40 + 38)
    h1 = _packed_head(fm1p, heads_1_w_mat, heads_1_bias, 24, m1)
    h2 = _packed_head(s2p, heads_2_w_mat, heads_2_bias, 16, m2)
    h3 = _conv_mm(s3, heads_3_w_mat, heads_3_bias, False, jnp.float32, 48,
                  stride=1, pad=1)
    h4 = _conv_mm(s4, heads_4_w_mat, heads_4_bias, False, jnp.float32, 32,
                  stride=1, pad=1)
    h5 = _matmul_bias(s5.reshape(B, 256), heads_5_w_mat[4 * 256:5 * 256, :],
                      heads_5_bias, False, jnp.float32)

    locs, confs = [], []
    for args in ((h0, 38, 38, 40, 24), (h1, 19, 19, 24, 24),
                 (h2, 10, 10, 16, 24)):
        l, c = _head_unpack(*args)
        locs.append(l)
        confs.append(c)
    for y, nl in ((h3, 24), (h4, 16)):
        locs.append(y[..., :nl].reshape(B, -1))
        confs.append(y[..., nl:2 * nl].reshape(B, -1))
    locs.append(h5[:, :16].reshape(B, -1))
    confs.append(h5[:, 16:32].reshape(B, -1))
    loc = jnp.concatenate(locs, axis=1).reshape(B, -1, 4)
    conf = jnp.concatenate(confs, axis=1).reshape(B, -1, 4)
    return loc, conf


# speedup vs baseline: 5.4334x; 1.0424x over previous
import functools

import jax
import jax.numpy as jnp
from jax.experimental import pallas as pl
from jax.experimental.pallas import tpu as pltpu

_VMEM_LIMIT = 48 * 1024 * 1024


def _ru(x, m):
    return ((x + m - 1) // m) * m


def _taps_kernel(x_ref, w_ref, b_ref, o_ref, *, offsets, cin, m_out, relu,
                 pad_out=None):
    acc = None
    for t, off in enumerate(offsets):
        part = jnp.dot(x_ref[pl.ds(off, m_out), :],
                       w_ref[pl.ds(t * cin, cin), :],
                       preferred_element_type=jnp.float32)
        acc = part if acc is None else acc + part
    y = acc + b_ref[...]
    if relu:
        y = jnp.maximum(y, 0.0)
    if pad_out is None:
        o_ref[...] = y.astype(o_ref.dtype)
    else:
        shift, wp, wlim = pad_out
        i = jax.lax.broadcasted_iota(jnp.int32, (m_out, 1), 0)
        ow = i - (i // wp) * wp
        y = jnp.where(ow < wlim, y, 0.0)
        o_ref[...] = jnp.zeros(o_ref.shape, o_ref.dtype)
        o_ref[pl.ds(shift, m_out), :] = y.astype(o_ref.dtype)


def _packed_taps_kernel(x_ref, w_ref, b_ref, o_ref, *, offsets, m_out):
    mread = offsets[-1] + m_out
    p = jnp.dot(x_ref[pl.ds(0, mread), :], w_ref[...],
                preferred_element_type=jnp.float32)
    acc = None
    for t, off in enumerate(offsets):
        sl = p[off:off + m_out, t * 128:(t + 1) * 128]
        acc = sl if acc is None else acc + sl
    o_ref[...] = (acc + b_ref[...]).astype(o_ref.dtype)


def _mm_kernel(x_ref, w_ref, b_ref, o_ref, *, relu):
    y = jnp.dot(x_ref[...], w_ref[...],
                preferred_element_type=jnp.float32) + b_ref[...]
    if relu:
        y = jnp.maximum(y, 0.0)
    o_ref[...] = y.astype(o_ref.dtype)


def _fused_conv(xp, w_mat, bias, offsets, m_out, relu, out_dtype,
                pad_out=None):
    B, R, C = xp.shape
    Np = w_mat.shape[1]
    TN = Np if Np <= 1024 else 512
    rows_out = m_out if pad_out is None else pad_out[0]
    po = None if pad_out is None else pad_out[1:]
    out = pl.pallas_call(
        functools.partial(_taps_kernel, offsets=offsets, cin=C,
                          m_out=m_out, relu=relu, pad_out=po),
        out_shape=jax.ShapeDtypeStruct((B, rows_out, Np), out_dtype),
        grid=(Np // TN, B),
        in_specs=[
            pl.BlockSpec((None, R, C), lambda j, b: (b, 0, 0)),
            pl.BlockSpec((len(offsets) * C, TN), lambda j, b: (0, j)),
            pl.BlockSpec((1, TN), lambda j, b: (0, j)),
        ],
        out_specs=pl.BlockSpec((None, rows_out, TN), lambda j, b: (b, 0, j)),
        compiler_params=pltpu.CompilerParams(
            dimension_semantics=("parallel", "parallel"),
            vmem_limit_bytes=_VMEM_LIMIT),
    )(xp, w_mat, bias)
    return out


def _packed_head(xp, w_mat, bias, Wp, m_out):
    B, R, C = xp.shape
    w_pack = w_mat.reshape(9, C, 128).transpose(1, 0, 2).reshape(C, 9 * 128)
    offsets = _s1_offsets(Wp)
    out = pl.pallas_call(
        functools.partial(_packed_taps_kernel, offsets=offsets, m_out=m_out),
        out_shape=jax.ShapeDtypeStruct((B, m_out, 128), jnp.float32),
        grid=(B,),
        in_specs=[
            pl.BlockSpec((None, R, C), lambda b: (b, 0, 0)),
            pl.BlockSpec((C, 9 * 128), lambda b: (0, 0)),
            pl.BlockSpec((1, 128), lambda b: (0, 0)),
        ],
        out_specs=pl.BlockSpec((None, m_out, 128), lambda b: (b, 0, 0)),
        compiler_params=pltpu.CompilerParams(
            dimension_semantics=("parallel",),
            vmem_limit_bytes=_VMEM_LIMIT),
    )(xp, w_pack, bias)
    return out


def _matmul_bias(x, w_mat, bias, relu, out_dtype):
    M, K = x.shape
    Np = w_mat.shape[1]
    if M >= 4096:
        TM = 512
        for c in (512, 448, 384, 320, 256):
            if M % c == 0:
                TM = c
                break
    elif M >= 128:
        TM = _ru((M + 1) // 2, 16)
    else:
        TM = _ru(M, 16)
    Mp = _ru(M, TM)
    if Mp != M:
        x = jnp.pad(x, ((0, Mp - M), (0, 0)))
    out = pl.pallas_call(
        functools.partial(_mm_kernel, relu=relu),
        out_shape=jax.ShapeDtypeStruct((Mp, Np), out_dtype),
        grid=(Mp // TM,),
        in_specs=[
            pl.BlockSpec((TM, K), lambda i: (i, 0)),
            pl.BlockSpec((K, Np), lambda i: (0, 0)),
            pl.BlockSpec((1, Np), lambda i: (0, 0)),
        ],
        out_specs=pl.BlockSpec((TM, Np), lambda i: (i, 0)),
        compiler_params=pltpu.CompilerParams(
            dimension_semantics=("parallel",),
            vmem_limit_bytes=_VMEM_LIMIT),
    )(x, w_mat, bias)
    if Mp != M:
        out = out[:M]
    return out


def _s1_offsets(Wp):
    return tuple(di * Wp + dj for di in range(3) for dj in range(3))


def _parity_planes(xpad):
    B, He, We, C = xpad.shape
    PH, PW = He // 2, We // 2
    PWp = _ru(PW, 8)
    xp = xpad.reshape(B, PH, 2, PW, 2, C).transpose(0, 2, 4, 1, 3, 5)
    xp = jnp.pad(xp, ((0, 0), (0, 0), (0, 0), (0, 0), (0, PWp - PW), (0, 0)))
    S = PH * PWp
    xp = xp.reshape(B, 4 * S, C)
    offsets = tuple(((di % 2) * 2 + (dj % 2)) * S + (di // 2) * PWp + (dj // 2)
                    for di in range(3) for dj in range(3))
    return xp, offsets, PWp


def _im2col3(x, stride, pad):
    B, H, W, C = x.shape
    if pad:
        x = jnp.pad(x, ((0, 0), (pad, pad), (pad, pad), (0, 0)))
    OH = (H + 2 * pad - 3) // stride + 1
    OW = (W + 2 * pad - 3) // stride + 1
    cols = [x[:, i:i + stride * OH:stride, j:j + stride * OW:stride, :]
            for i in range(3) for j in range(3)]
    return jnp.concatenate(cols, -1).reshape(B * OH * OW, 9 * C), OH, OW


def _conv_mm(x, w_mat, bias, relu, out_dtype, n, stride=1, pad=0, k=3):
    B = x.shape[0]
    if k == 1:
        patches = x.reshape(-1, x.shape[-1])
        OH, OW = x.shape[1], x.shape[2]
    else:
        patches, OH, OW = _im2col3(x, stride, pad)
    y = _matmul_bias(patches, w_mat, bias, relu, out_dtype)
    return y.reshape(B, OH, OW, -1)[..., :n]


def _head_unpack(y, OH, OW, Wp, nl):
    B, m, _ = y.shape
    y = y[..., :2 * nl]
    y = jnp.pad(y, ((0, 0), (0, OH * Wp - m), (0, 0)))
    y = y.reshape(B, OH, Wp, 2 * nl)[:, :, :OW, :]
    return y[..., :nl].reshape(B, -1), y[..., nl:].reshape(B, -1)


def kernel(x_nchw, base_0_w_mat, base_0_bias, base_1_w_mat, base_1_bias,
           extras_0_w_mat, extras_0_bias, extras_1_w_mat, extras_1_bias,
           extras_2_w_mat, extras_2_bias, extras_3_w_mat, extras_3_bias,
           extras_4_w_mat, extras_4_bias, extras_5_w_mat, extras_5_bias,
           extras_6_w_mat, extras_6_bias, extras_7_w_mat, extras_7_bias,
           heads_0_w_mat, heads_0_bias, heads_1_w_mat, heads_1_bias,
           heads_2_w_mat, heads_2_bias, heads_3_w_mat, heads_3_bias,
           heads_4_w_mat, heads_4_bias, heads_5_w_mat, heads_5_bias):
    B = x_nchw.shape[0]
    x = jnp.transpose(x_nchw, (0, 2, 3, 1)).astype(jnp.bfloat16)

    H, W = x.shape[1], x.shape[2]
    Hp, Wp0 = H + 2, W + 2
    x0 = jnp.pad(x, ((0, 0), (1, 1), (1, 1), (0, 125))).reshape(B, 1600, 128)
    w0 = jnp.pad(base_0_w_mat.reshape(9, 3, 512),
                 ((0, 0), (0, 125), (0, 0))).reshape(9 * 128, 512)
    m0 = (38 - 1) * 40 + 38
    fm0p = _fused_conv(x0, w0, base_0_bias, _s1_offsets(40), m0, True,
                       jnp.bfloat16, pad_out=(1600, 41, 40, 38))
    fm0p = fm0p.reshape(B, Hp, Wp0, 512)

    xp1, offs1, PWp1 = _parity_planes(fm0p)
    m1 = (19 - 1) * 24 + 19
    fm1p = _fused_conv(xp1, base_1_w_mat, base_1_bias, offs1, m1, True,
                       jnp.bfloat16, pad_out=(21 * 24, 24 + 1, 24, 19))

    e0 = _matmul_bias(fm1p.reshape(B * 504, 1024), extras_0_w_mat,
                      extras_0_bias, True, jnp.bfloat16)
    e0 = e0.reshape(B, 21, 24, 256)[:, 1:20, 1:20, :]
    e0s = jnp.pad(e0, ((0, 0), (1, 2), (1, 2), (0, 0)))

    xp2, offs2, PWp2 = _parity_planes(e0s)
    m2 = (10 - 1) * 16 + 10
    s2p = _fused_conv(xp2, extras_1_w_mat, extras_1_bias, offs2, m2, True,
                      jnp.bfloat16, pad_out=(12 * 16, 16 + 1, 16, 10))

    e2 = _matmul_bias(s2p.reshape(B * 192, 512), extras_2_w_mat,
                      extras_2_bias, True, jnp.bfloat16)
    e2s = e2.reshape(B, 12, 16, 128)[:, 1:11, 1:11, :]

    s3 = _conv_mm(e2s, extras_3_w_mat, extras_3_bias, True, jnp.bfloat16, 256,
                  stride=2, pad=1)
    e4 = _conv_mm(s3, extras_4_w_mat, extras_4_bias, True, jnp.bfloat16, 128,
                  k=1)
    s4 = _conv_mm(e4, extras_5_w_mat, extras_5_bias, True, jnp.bfloat16, 256,
                  stride=1, pad=0)
    e6 = _conv_mm(s4, extras_6_w_mat, extras_6_bias, True, jnp.bfloat16, 128,
                  k=1)
    s5 = _conv_mm(e6, extras_7_w_mat, extras_7_bias, True, jnp.bfloat16, 256,
                  stride=1, pad=0)

    h0 = _packed_head(fm0p.reshape(B, Hp * Wp0, 512), heads_0_w_mat,
                      heads_0_bias, 40, (38 - 1) * 40 + 38)
    h1 = _packed_head(fm1p, heads_1_w_mat, heads_1_bias, 24, m1)
    h2 = _packed_head(s2p, heads_2_w_mat, heads_2_bias, 16, m2)
    h3 = _conv_mm(s3, heads_3_w_mat, heads_3_bias, False, jnp.float32, 48,
                  stride=1, pad=1)
    h4 = _conv_mm(s4, heads_4_w_mat, heads_4_bias, False, jnp.float32, 32,
                  stride=1, pad=1)
    h5 = _matmul_bias(s5.reshape(B, 256), heads_5_w_mat[4 * 256:5 * 256, :],
                      heads_5_bias, False, jnp.float32)

    locs, confs = [], []
    for args in ((h0, 38, 38, 40, 24), (h1, 19, 19, 24, 24),
                 (h2, 10, 10, 16, 24)):
        l, c = _head_unpack(*args)
        locs.append(l)
        confs.append(c)
    for y, nl in ((h3, 24), (h4, 16)):
        locs.append(y[..., :nl].reshape(B, -1))
        confs.append(y[..., nl:2 * nl].reshape(B, -1))
    locs.append(h5[:, :16].reshape(B, -1))
    confs.append(h5[:, 16:32].reshape(B, -1))
    loc = jnp.concatenate(locs, axis=1).reshape(B, -1, 4)
    conf = jnp.concatenate(confs, axis=1).reshape(B, -1, 4)
    return loc, conf


# speedup vs baseline: 5.8651x; 1.0795x over previous
import functools

import jax
import jax.numpy as jnp
from jax.experimental import pallas as pl
from jax.experimental.pallas import tpu as pltpu

_VMEM_LIMIT = 48 * 1024 * 1024


def _ru(x, m):
    return ((x + m - 1) // m) * m


def _taps_kernel(x_ref, w_ref, b_ref, o_ref, *, offsets, cin, m_out, relu,
                 pad_out=None):
    acc = None
    for t, off in enumerate(offsets):
        part = jnp.dot(x_ref[pl.ds(off, m_out), :],
                       w_ref[pl.ds(t * cin, cin), :],
                       preferred_element_type=jnp.float32)
        acc = part if acc is None else acc + part
    y = acc + b_ref[...]
    if relu:
        y = jnp.maximum(y, 0.0)
    if pad_out is None:
        o_ref[...] = y.astype(o_ref.dtype)
    else:
        shift, wp, wlim = pad_out
        i = jax.lax.broadcasted_iota(jnp.int32, (m_out, 1), 0)
        ow = i - (i // wp) * wp
        y = jnp.where(ow < wlim, y, 0.0)
        o_ref[...] = jnp.zeros(o_ref.shape, o_ref.dtype)
        o_ref[pl.ds(shift, m_out), :] = y.astype(o_ref.dtype)


def _packed_taps_kernel(x_ref, w_ref, b_ref, o_ref, *, offsets, m_out):
    mread = offsets[-1] + m_out
    p = jnp.dot(x_ref[pl.ds(0, mread), :], w_ref[...],
                preferred_element_type=jnp.float32)
    acc = None
    for t, off in enumerate(offsets):
        sl = p[off:off + m_out, t * 128:(t + 1) * 128]
        acc = sl if acc is None else acc + sl
    o_ref[...] = (acc + b_ref[...]).astype(o_ref.dtype)


def _base0_kernel(x_ref, w_ref, b_ref, o_ref):
    y = jnp.dot(x_ref[...], w_ref[...],
                preferred_element_type=jnp.float32) + b_ref[...]
    y = jnp.maximum(y, 0.0)
    i = jax.lax.broadcasted_iota(jnp.int32, (1600, 1), 0)
    pr = i // 40
    pc = i - pr * 40
    valid = (pr >= 1) & (pr <= 38) & (pc >= 1) & (pc <= 38)
    o_ref[...] = jnp.where(valid, y, 0.0).astype(o_ref.dtype)


def _mm_kernel(x_ref, w_ref, b_ref, o_ref, *, relu):
    y = jnp.dot(x_ref[...], w_ref[...],
                preferred_element_type=jnp.float32) + b_ref[...]
    if relu:
        y = jnp.maximum(y, 0.0)
    o_ref[...] = y.astype(o_ref.dtype)


def _fused_conv(xp, w_mat, bias, offsets, m_out, relu, out_dtype,
                pad_out=None):
    B, R, C = xp.shape
    Np = w_mat.shape[1]
    TN = Np if Np <= 1024 else 512
    rows_out = m_out if pad_out is None else pad_out[0]
    po = None if pad_out is None else pad_out[1:]
    out = pl.pallas_call(
        functools.partial(_taps_kernel, offsets=offsets, cin=C,
                          m_out=m_out, relu=relu, pad_out=po),
        out_shape=jax.ShapeDtypeStruct((B, rows_out, Np), out_dtype),
        grid=(Np // TN, B),
        in_specs=[
            pl.BlockSpec((None, R, C), lambda j, b: (b, 0, 0)),
            pl.BlockSpec((len(offsets) * C, TN), lambda j, b: (0, j)),
            pl.BlockSpec((1, TN), lambda j, b: (0, j)),
        ],
        out_specs=pl.BlockSpec((None, rows_out, TN), lambda j, b: (b, 0, j)),
        compiler_params=pltpu.CompilerParams(
            dimension_semantics=("parallel", "parallel"),
            vmem_limit_bytes=_VMEM_LIMIT),
    )(xp, w_mat, bias)
    return out


def _packed_head(xp, w_mat, bias, Wp, m_out):
    B, R, C = xp.shape
    w_pack = w_mat.reshape(9, C, 128).transpose(1, 0, 2).reshape(C, 9 * 128)
    offsets = _s1_offsets(Wp)
    out = pl.pallas_call(
        functools.partial(_packed_taps_kernel, offsets=offsets, m_out=m_out),
        out_shape=jax.ShapeDtypeStruct((B, m_out, 128), jnp.float32),
        grid=(B,),
        in_specs=[
            pl.BlockSpec((None, R, C), lambda b: (b, 0, 0)),
            pl.BlockSpec((C, 9 * 128), lambda b: (0, 0)),
            pl.BlockSpec((1, 128), lambda b: (0, 0)),
        ],
        out_specs=pl.BlockSpec((None, m_out, 128), lambda b: (b, 0, 0)),
        compiler_params=pltpu.CompilerParams(
            dimension_semantics=("parallel",),
            vmem_limit_bytes=_VMEM_LIMIT),
    )(xp, w_pack, bias)
    return out


def _matmul_bias(x, w_mat, bias, relu, out_dtype):
    M, K = x.shape
    Np = w_mat.shape[1]
    if M >= 4096:
        TM = 512
        for c in (512, 448, 384, 320, 256):
            if M % c == 0:
                TM = c
                break
    elif M >= 128:
        TM = _ru((M + 1) // 2, 16)
    else:
        TM = _ru(M, 16)
    Mp = _ru(M, TM)
    if Mp != M:
        x = jnp.pad(x, ((0, Mp - M), (0, 0)))
    out = pl.pallas_call(
        functools.partial(_mm_kernel, relu=relu),
        out_shape=jax.ShapeDtypeStruct((Mp, Np), out_dtype),
        grid=(Mp // TM,),
        in_specs=[
            pl.BlockSpec((TM, K), lambda i: (i, 0)),
            pl.BlockSpec((K, Np), lambda i: (0, 0)),
            pl.BlockSpec((1, Np), lambda i: (0, 0)),
        ],
        out_specs=pl.BlockSpec((TM, Np), lambda i: (i, 0)),
        compiler_params=pltpu.CompilerParams(
            dimension_semantics=("parallel",),
            vmem_limit_bytes=_VMEM_LIMIT),
    )(x, w_mat, bias)
    if Mp != M:
        out = out[:M]
    return out


def _s1_offsets(Wp):
    return tuple(di * Wp + dj for di in range(3) for dj in range(3))


def _parity_planes(xpad):
    B, He, We, C = xpad.shape
    PH, PW = He // 2, We // 2
    PWp = _ru(PW, 8)
    xp = xpad.reshape(B, PH, 2, PW, 2, C).transpose(0, 2, 4, 1, 3, 5)
    xp = jnp.pad(xp, ((0, 0), (0, 0), (0, 0), (0, 0), (0, PWp - PW), (0, 0)))
    S = PH * PWp
    xp = xp.reshape(B, 4 * S, C)
    offsets = tuple(((di % 2) * 2 + (dj % 2)) * S + (di // 2) * PWp + (dj // 2)
                    for di in range(3) for dj in range(3))
    return xp, offsets, PWp


def _im2col3(x, stride, pad):
    B, H, W, C = x.shape
    if pad:
        x = jnp.pad(x, ((0, 0), (pad, pad), (pad, pad), (0, 0)))
    OH = (H + 2 * pad - 3) // stride + 1
    OW = (W + 2 * pad - 3) // stride + 1
    cols = [x[:, i:i + stride * OH:stride, j:j + stride * OW:stride, :]
            for i in range(3) for j in range(3)]
    return jnp.concatenate(cols, -1).reshape(B * OH * OW, 9 * C), OH, OW


def _conv_mm(x, w_mat, bias, relu, out_dtype, n, stride=1, pad=0, k=3):
    B = x.shape[0]
    if k == 1:
        patches = x.reshape(-1, x.shape[-1])
        OH, OW = x.shape[1], x.shape[2]
    else:
        patches, OH, OW = _im2col3(x, stride, pad)
    y = _matmul_bias(patches, w_mat, bias, relu, out_dtype)
    return y.reshape(B, OH, OW, -1)[..., :n]


def _head_unpack(y, OH, OW, Wp, nl):
    B, m, _ = y.shape
    y = y[..., :2 * nl]
    y = jnp.pad(y, ((0, 0), (0, OH * Wp - m), (0, 0)))
    y = y.reshape(B, OH, Wp, 2 * nl)[:, :, :OW, :]
    return y[..., :nl].reshape(B, -1), y[..., nl:].reshape(B, -1)


def kernel(x_nchw, base_0_w_mat, base_0_bias, base_1_w_mat, base_1_bias,
           extras_0_w_mat, extras_0_bias, extras_1_w_mat, extras_1_bias,
           extras_2_w_mat, extras_2_bias, extras_3_w_mat, extras_3_bias,
           extras_4_w_mat, extras_4_bias, extras_5_w_mat, extras_5_bias,
           extras_6_w_mat, extras_6_bias, extras_7_w_mat, extras_7_bias,
           heads_0_w_mat, heads_0_bias, heads_1_w_mat, heads_1_bias,
           heads_2_w_mat, heads_2_bias, heads_3_w_mat, heads_3_bias,
           heads_4_w_mat, heads_4_bias, heads_5_w_mat, heads_5_bias):
    B = x_nchw.shape[0]
    x = jnp.transpose(x_nchw, (0, 2, 3, 1)).astype(jnp.bfloat16)

    H, W = x.shape[1], x.shape[2]
    Hp, Wp0 = H + 2, W + 2
    xp2 = jnp.pad(x, ((0, 0), (2, 2), (2, 2), (0, 0)))
    patches = jnp.concatenate(
        [xp2[:, i:i + Hp, j:j + Wp0, :] for i in range(3) for j in range(3)],
        -1)
    x0 = jnp.pad(patches, ((0, 0), (0, 0), (0, 0), (0, 101)))
    x0 = x0.reshape(B, 1600, 128)
    w0 = jnp.pad(base_0_w_mat, ((0, 101), (0, 0)))
    fm0p = pl.pallas_call(
        _base0_kernel,
        out_shape=jax.ShapeDtypeStruct((B, 1600, 512), jnp.bfloat16),
        grid=(B,),
        in_specs=[
            pl.BlockSpec((None, 1600, 128), lambda b: (b, 0, 0)),
            pl.BlockSpec((128, 512), lambda b: (0, 0)),
            pl.BlockSpec((1, 512), lambda b: (0, 0)),
        ],
        out_specs=pl.BlockSpec((None, 1600, 512), lambda b: (b, 0, 0)),
        compiler_params=pltpu.CompilerParams(
            dimension_semantics=("parallel",),
            vmem_limit_bytes=_VMEM_LIMIT),
    )(x0, w0, base_0_bias)
    fm0p = fm0p.reshape(B, Hp, Wp0, 512)

    xp1, offs1, PWp1 = _parity_planes(fm0p)
    m1 = (19 - 1) * 24 + 19
    fm1p = _fused_conv(xp1, base_1_w_mat, base_1_bias, offs1, m1, True,
                       jnp.bfloat16, pad_out=(21 * 24, 24 + 1, 24, 19))

    e0 = _matmul_bias(fm1p.reshape(B * 504, 1024), extras_0_w_mat,
                      extras_0_bias, True, jnp.bfloat16)
    e0 = e0.reshape(B, 21, 24, 256)[:, 1:20, 1:20, :]
    e0s = jnp.pad(e0, ((0, 0), (1, 2), (1, 2), (0, 0)))

    xp2, offs2, PWp2 = _parity_planes(e0s)
    m2 = (10 - 1) * 16 + 10
    s2p = _fused_conv(xp2, extras_1_w_mat, extras_1_bias, offs2, m2, True,
                      jnp.bfloat16, pad_out=(12 * 16, 16 + 1, 16, 10))

    e2 = _matmul_bias(s2p.reshape(B * 192, 512), extras_2_w_mat,
                      extras_2_bias, True, jnp.bfloat16)
    e2s = e2.reshape(B, 12, 16, 128)[:, 1:11, 1:11, :]

    s3 = _conv_mm(e2s, extras_3_w_mat, extras_3_bias, True, jnp.bfloat16, 256,
                  stride=2, pad=1)
    e4 = _conv_mm(s3, extras_4_w_mat, extras_4_bias, True, jnp.bfloat16, 128,
                  k=1)
    s4 = _conv_mm(e4, extras_5_w_mat, extras_5_bias, True, jnp.bfloat16, 256,
                  stride=1, pad=0)
    e6 = _conv_mm(s4, extras_6_w_mat, extras_6_bias, True, jnp.bfloat16, 128,
                  k=1)
    s5 = _conv_mm(e6, extras_7_w_mat, extras_7_bias, True, jnp.bfloat16, 256,
                  stride=1, pad=0)

    h0 = _packed_head(fm0p.reshape(B, Hp * Wp0, 512), heads_0_w_mat,
                      heads_0_bias, 40, (38 - 1) * 40 + 38)
    h1 = _packed_head(fm1p, heads_1_w_mat, heads_1_bias, 24, m1)
    h2 = _packed_head(s2p, heads_2_w_mat, heads_2_bias, 16, m2)
    h3 = _conv_mm(s3, heads_3_w_mat, heads_3_bias, False, jnp.float32, 48,
                  stride=1, pad=1)
    h4 = _conv_mm(s4, heads_4_w_mat, heads_4_bias, False, jnp.float32, 32,
                  stride=1, pad=1)
    h5 = _matmul_bias(s5.reshape(B, 256), heads_5_w_mat[4 * 256:5 * 256, :],
                      heads_5_bias, False, jnp.float32)

    locs, confs = [], []
    for args in ((h0, 38, 38, 40, 24), (h1, 19, 19, 24, 24),
                 (h2, 10, 10, 16, 24)):
        l, c = _head_unpack(*args)
        locs.append(l)
        confs.append(c)
    for y, nl in ((h3, 24), (h4, 16)):
        locs.append(y[..., :nl].reshape(B, -1))
        confs.append(y[..., nl:2 * nl].reshape(B, -1))
    locs.append(h5[:, :16].reshape(B, -1))
    confs.append(h5[:, 16:32].reshape(B, -1))
    loc = jnp.concatenate(locs, axis=1).reshape(B, -1, 4)
    conf = jnp.concatenate(confs, axis=1).reshape(B, -1, 4)
    return loc, conf
